# baseline (speedup 1.0000x reference)
import functools
import math

import jax
import jax.numpy as jnp
import numpy as np
from jax.experimental import pallas as pl

LEVEL_SIZES = [(100, 100), (50, 50), (25, 25), (13, 13), (7, 7)]
A = 3
IMG_H = 800.0
IMG_W = 800.0
PRE_NMS_TOP_N = 1000
POST_NMS_TOP_N = 1000
NMS_THRESH = 0.7
MIN_SIZE = 1e-3
BBOX_XFORM_CLIP = float(np.log(1000.0 / 16.0))
NUM_PER_LEVEL = [h * w * A for (h, w) in LEVEL_SIZES]
N_TOTAL = sum(NUM_PER_LEVEL)


def _decode(deltas, anchors):
    w = anchors[:, 2] - anchors[:, 0]
    h = anchors[:, 3] - anchors[:, 1]
    cx = anchors[:, 0] + 0.5 * w
    cy = anchors[:, 1] + 0.5 * h
    dx = deltas[..., 0]
    dy = deltas[..., 1]
    dw = jnp.minimum(deltas[..., 2], BBOX_XFORM_CLIP)
    dh = jnp.minimum(deltas[..., 3], BBOX_XFORM_CLIP)
    pcx = dx * w + cx
    pcy = dy * h + cy
    pw = jnp.exp(dw) * w
    ph = jnp.exp(dh) * h
    return jnp.stack([pcx - 0.5 * pw, pcy - 0.5 * ph, pcx + 0.5 * pw, pcy + 0.5 * ph], axis=-1)


def _box_iou(b):
    area = (b[:, 2] - b[:, 0]) * (b[:, 3] - b[:, 1])
    lt = jnp.maximum(b[:, None, :2], b[None, :, :2])
    rb = jnp.minimum(b[:, None, 2:], b[None, :, 2:])
    wh = jnp.maximum(rb - lt, 0.0)
    inter = wh[..., 0] * wh[..., 1]
    return inter / (area[:, None] + area[None, :] - inter + 1e-9)


def _nms_keep(boxes_sorted, thresh):
    K = boxes_sorted.shape[0]
    iou = _box_iou(boxes_sorted)
    ar = jnp.arange(K)

    def body(i, keep):
        sup = (iou[i] > thresh) & (ar > i) & keep[i]
        return keep & (~sup)

    return jax.lax.fori_loop(0, K, body, jnp.ones((K,), dtype=bool))


def _identity_kernel(x_ref, o_ref):
    o_ref[...] = x_ref[...]


def kernel(objectness, pred_bbox_deltas, anchors):
    proposals = _decode(pred_bbox_deltas, anchors)
    levels = jnp.concatenate(
        [jnp.full((n,), i, dtype=jnp.int32) for i, n in enumerate(NUM_PER_LEVEL)])
    idx_list = []
    offset = 0
    for n in NUM_PER_LEVEL:
        k = min(PRE_NMS_TOP_N, n)
        _, ti = jax.lax.top_k(objectness[:, offset:offset + n], k)
        idx_list.append(ti + offset)
        offset += n
    top_idx = jnp.concatenate(idx_list, axis=1)
    B = objectness.shape[0]
    bi = jnp.arange(B)[:, None]
    sc = objectness[bi, top_idx]
    bx = proposals[bi, top_idx]
    lv = levels[top_idx]
    outs = []
    for b in range(B):
        boxes = bx[b]
        boxes = jnp.stack([
            jnp.clip(boxes[:, 0], 0.0, IMG_W),
            jnp.clip(boxes[:, 1], 0.0, IMG_H),
            jnp.clip(boxes[:, 2], 0.0, IMG_W),
            jnp.clip(boxes[:, 3], 0.0, IMG_H)], axis=1)
        ws = boxes[:, 2] - boxes[:, 0]
        hs = boxes[:, 3] - boxes[:, 1]
        valid = (ws >= MIN_SIZE) & (hs >= MIN_SIZE)
        s = jnp.where(valid, sc[b], -1e10)
        off = lv[b].astype(boxes.dtype) * (float(max(IMG_H, IMG_W)) + 1.0)
        boxes_off = boxes + off[:, None]
        order = jnp.argsort(-s)
        boxes_s = boxes_off[order]
        s_s = s[order]
        boxes_o = boxes[order]
        keep = _nms_keep(boxes_s, NMS_THRESH)
        masked = jnp.where(keep, s_s, -1e10)
        top_s, ti2 = jax.lax.top_k(masked, POST_NMS_TOP_N)
        out_b = boxes_o[ti2]
        outs.append(jnp.concatenate([out_b, top_s[:, None]], axis=1))
    out = jnp.stack(outs, axis=0)
    out = pl.pallas_call(
        _identity_kernel,
        out_shape=jax.ShapeDtypeStruct(out.shape, out.dtype),
    )(out)
    return out


# speedup vs baseline: 10.1373x; 10.1373x over previous
import functools

import jax
import jax.numpy as jnp
import numpy as np
from jax.experimental import pallas as pl
from jax.experimental.pallas import tpu as pltpu

LEVEL_SIZES = [(100, 100), (50, 50), (25, 25), (13, 13), (7, 7)]
A = 3
IMG_H = 800.0
IMG_W = 800.0
PRE_NMS_TOP_N = 1000
POST_NMS_TOP_N = 1000
NMS_THRESH = 0.7
MIN_SIZE = 1e-3
BBOX_XFORM_CLIP = float(np.log(1000.0 / 16.0))
NUM_PER_LEVEL = [h * w * A for (h, w) in LEVEL_SIZES]
N_TOTAL = sum(NUM_PER_LEVEL)
K_PER_LEVEL = [min(PRE_NMS_TOP_N, n) for n in NUM_PER_LEVEL]
TILE = 1024
NEG_PAD = -1e30


def _nms_tile_kernel(in_ref, out_ref, iou_ref):
    s = in_ref[0, 0, :]
    d0 = in_ref[0, 1, :]
    d1 = in_ref[0, 2, :]
    d2 = in_ref[0, 3, :]
    d3 = in_ref[0, 4, :]
    a0 = in_ref[0, 5, :]
    a1 = in_ref[0, 6, :]
    a2 = in_ref[0, 7, :]
    a3 = in_ref[0, 8, :]

    w = a2 - a0
    h = a3 - a1
    cx = a0 + 0.5 * w
    cy = a1 + 0.5 * h
    dw = jnp.minimum(d2, BBOX_XFORM_CLIP)
    dh = jnp.minimum(d3, BBOX_XFORM_CLIP)
    pcx = d0 * w + cx
    pcy = d1 * h + cy
    pw = jnp.exp(dw) * w
    ph = jnp.exp(dh) * h
    x1 = jnp.clip(pcx - 0.5 * pw, 0.0, IMG_W)
    y1 = jnp.clip(pcy - 0.5 * ph, 0.0, IMG_H)
    x2 = jnp.clip(pcx + 0.5 * pw, 0.0, IMG_W)
    y2 = jnp.clip(pcy + 0.5 * ph, 0.0, IMG_H)

    out_ref[0, 1, :] = x1
    out_ref[0, 2, :] = y1
    out_ref[0, 3, :] = x2
    out_ref[0, 4, :] = y2
    out_ref[0, 5, :] = jnp.zeros((TILE,), jnp.float32)
    out_ref[0, 6, :] = jnp.zeros((TILE,), jnp.float32)
    out_ref[0, 7, :] = jnp.zeros((TILE,), jnp.float32)

    valid = (x2 - x1 >= MIN_SIZE) & (y2 - y1 >= MIN_SIZE) & (s > -1e20)

    area = (x2 - x1) * (y2 - y1)
    lx = jnp.maximum(x1[:, None], x1[None, :])
    ly = jnp.maximum(y1[:, None], y1[None, :])
    rx = jnp.minimum(x2[:, None], x2[None, :])
    ry = jnp.minimum(y2[:, None], y2[None, :])
    iw = jnp.maximum(rx - lx, 0.0)
    ih = jnp.maximum(ry - ly, 0.0)
    inter = iw * ih
    iou = inter / (area[:, None] + area[None, :] - inter + 1e-9)
    ii = jax.lax.broadcasted_iota(jnp.int32, (TILE, TILE), 0)
    jj = jax.lax.broadcasted_iota(jnp.int32, (TILE, TILE), 1)
    sup = (iou > NMS_THRESH) & (jj > ii)
    iou_ref[...] = jnp.where(sup, 1.0, 0.0).reshape(TILE // 8, 8, TILE)

    lane = jax.lax.broadcasted_iota(jnp.int32, (1, TILE), 1)
    sub = jax.lax.broadcasted_iota(jnp.int32, (8, TILE), 0)

    def body(i, keep):
        slab = iou_ref[pl.ds(i // 8, 1), :, :]
        row = jnp.sum(jnp.where(sub == i % 8, slab.reshape(8, TILE), 0.0),
                      axis=0, keepdims=True)
        ki = jnp.sum(jnp.where(lane == i, keep, 0.0), keepdims=True)
        return keep * (1.0 - row * ki)

    keep_f = jax.lax.fori_loop(
        0, TILE, body, jnp.where(valid, 1.0, 0.0).reshape(1, TILE))

    keep = keep_f[0, :] > 0.5
    out_ref[0, 0, :] = jnp.where(
        keep, s, jnp.where(s > -1e20, -1e10, NEG_PAD))


@jax.jit
def kernel(objectness, pred_bbox_deltas, anchors):
    B = objectness.shape[0]
    nl = len(NUM_PER_LEVEL)

    sc_parts, dl_parts, an_parts = [], [], []
    offset = 0
    for lvl, n in enumerate(NUM_PER_LEVEL):
        k = K_PER_LEVEL[lvl]
        vals, ti = jax.lax.top_k(objectness[:, offset:offset + n], k)
        dl = jnp.take_along_axis(
            pred_bbox_deltas[:, offset:offset + n, :], ti[:, :, None], axis=1)
        an = anchors[offset:offset + n][ti]
        pad = TILE - k
        sc_parts.append(jnp.pad(vals, ((0, 0), (0, pad)),
                                constant_values=NEG_PAD))
        dl_parts.append(jnp.pad(dl, ((0, 0), (0, pad), (0, 0))))
        an_parts.append(jnp.pad(an, ((0, 0), (0, pad), (0, 0))))
        offset += n

    scores_p = jnp.stack(sc_parts, axis=1).reshape(B * nl, 1, TILE)
    deltas_p = jnp.stack(dl_parts, axis=1).reshape(B * nl, TILE, 4)
    anchors_p = jnp.stack(an_parts, axis=1).reshape(B * nl, TILE, 4)
    deltas_p = jnp.transpose(deltas_p, (0, 2, 1))
    anchors_p = jnp.transpose(anchors_p, (0, 2, 1))
    packed = jnp.concatenate(
        [scores_p, deltas_p, anchors_p,
         jnp.zeros((B * nl, 7, TILE), jnp.float32)], axis=1)

    out = pl.pallas_call(
        _nms_tile_kernel,
        grid=(B * nl,),
        in_specs=[pl.BlockSpec((1, 16, TILE), lambda i: (i, 0, 0))],
        out_specs=pl.BlockSpec((1, 8, TILE), lambda i: (i, 0, 0)),
        out_shape=jax.ShapeDtypeStruct((B * nl, 8, TILE), jnp.float32),
        scratch_shapes=[
            pltpu.VMEM((TILE // 8, 8, TILE), jnp.float32),
        ],
        compiler_params=pltpu.CompilerParams(
            dimension_semantics=("arbitrary",)),
    )(packed)

    masked = out[:, 0, :].reshape(B, nl * TILE)
    boxes = jnp.transpose(out[:, 1:5, :].reshape(B, nl, 4, TILE), (0, 1, 3, 2))
    boxes = boxes.reshape(B, nl * TILE, 4)

    top_s, ti2 = jax.lax.top_k(masked, POST_NMS_TOP_N)
    out_b = jnp.take_along_axis(boxes, ti2[:, :, None], axis=1)
    return jnp.concatenate([out_b, top_s[:, :, None]], axis=2)


# speedup vs baseline: 10.2147x; 1.0076x over previous
import functools

import jax
import jax.numpy as jnp
import numpy as np
from jax.experimental import pallas as pl
from jax.experimental.pallas import tpu as pltpu

LEVEL_SIZES = [(100, 100), (50, 50), (25, 25), (13, 13), (7, 7)]
A = 3
IMG_H = 800.0
IMG_W = 800.0
PRE_NMS_TOP_N = 1000
POST_NMS_TOP_N = 1000
NMS_THRESH = 0.7
MIN_SIZE = 1e-3
BBOX_XFORM_CLIP = float(np.log(1000.0 / 16.0))
NUM_PER_LEVEL = [h * w * A for (h, w) in LEVEL_SIZES]
N_TOTAL = sum(NUM_PER_LEVEL)
K_PER_LEVEL = [min(PRE_NMS_TOP_N, n) for n in NUM_PER_LEVEL]
TILE = 1024
NEG_PAD = -1e30


def _nms_tile_kernel(in_ref, out_ref, iou_ref):
    s = in_ref[0, 0, :]
    d0 = in_ref[0, 1, :]
    d1 = in_ref[0, 2, :]
    d2 = in_ref[0, 3, :]
    d3 = in_ref[0, 4, :]
    a0 = in_ref[0, 5, :]
    a1 = in_ref[0, 6, :]
    a2 = in_ref[0, 7, :]
    a3 = in_ref[0, 8, :]

    w = a2 - a0
    h = a3 - a1
    cx = a0 + 0.5 * w
    cy = a1 + 0.5 * h
    dw = jnp.minimum(d2, BBOX_XFORM_CLIP)
    dh = jnp.minimum(d3, BBOX_XFORM_CLIP)
    pcx = d0 * w + cx
    pcy = d1 * h + cy
    pw = jnp.exp(dw) * w
    ph = jnp.exp(dh) * h
    x1 = jnp.clip(pcx - 0.5 * pw, 0.0, IMG_W)
    y1 = jnp.clip(pcy - 0.5 * ph, 0.0, IMG_H)
    x2 = jnp.clip(pcx + 0.5 * pw, 0.0, IMG_W)
    y2 = jnp.clip(pcy + 0.5 * ph, 0.0, IMG_H)

    out_ref[0, 1, :] = x1
    out_ref[0, 2, :] = y1
    out_ref[0, 3, :] = x2
    out_ref[0, 4, :] = y2
    out_ref[0, 5, :] = jnp.zeros((TILE,), jnp.float32)
    out_ref[0, 6, :] = jnp.zeros((TILE,), jnp.float32)
    out_ref[0, 7, :] = jnp.zeros((TILE,), jnp.float32)

    valid = (x2 - x1 >= MIN_SIZE) & (y2 - y1 >= MIN_SIZE) & (s > -1e20)

    area = (x2 - x1) * (y2 - y1)
    lx = jnp.maximum(x1[:, None], x1[None, :])
    ly = jnp.maximum(y1[:, None], y1[None, :])
    rx = jnp.minimum(x2[:, None], x2[None, :])
    ry = jnp.minimum(y2[:, None], y2[None, :])
    iw = jnp.maximum(rx - lx, 0.0)
    ih = jnp.maximum(ry - ly, 0.0)
    inter = iw * ih
    iou = inter / (area[:, None] + area[None, :] - inter + 1e-9)
    ii = jax.lax.broadcasted_iota(jnp.int32, (TILE, TILE), 0)
    jj = jax.lax.broadcasted_iota(jnp.int32, (TILE, TILE), 1)
    sup = (iou > NMS_THRESH) & (jj > ii)
    iou_ref[...] = jnp.where(sup, 1.0, 0.0).reshape(TILE // 8, 8, TILE)

    lane = jax.lax.broadcasted_iota(jnp.int32, (1, TILE), 1)

    def body(q, keep):
        slab = iou_ref[pl.ds(q, 1), :, :].reshape(8, TILE)
        base = q * 8
        for r in range(8):
            row = slab[r:r + 1, :]
            ki = jnp.sum(jnp.where(lane == base + r, keep, 0.0),
                         keepdims=True)
            keep = keep * (1.0 - row * ki)
        return keep

    keep_f = jax.lax.fori_loop(
        0, TILE // 8, body, jnp.where(valid, 1.0, 0.0).reshape(1, TILE))

    keep = keep_f[0, :] > 0.5
    out_ref[0, 0, :] = jnp.where(
        keep, s, jnp.where(s > -1e20, -1e10, NEG_PAD))


@jax.jit
def kernel(objectness, pred_bbox_deltas, anchors):
    B = objectness.shape[0]
    nl = len(NUM_PER_LEVEL)

    sc_parts, dl_parts, an_parts = [], [], []
    offset = 0
    for lvl, n in enumerate(NUM_PER_LEVEL):
        k = K_PER_LEVEL[lvl]
        vals, ti = jax.lax.top_k(objectness[:, offset:offset + n], k)
        dl = jnp.take_along_axis(
            pred_bbox_deltas[:, offset:offset + n, :], ti[:, :, None], axis=1)
        an = anchors[offset:offset + n][ti]
        pad = TILE - k
        sc_parts.append(jnp.pad(vals, ((0, 0), (0, pad)),
                                constant_values=NEG_PAD))
        dl_parts.append(jnp.pad(dl, ((0, 0), (0, pad), (0, 0))))
        an_parts.append(jnp.pad(an, ((0, 0), (0, pad), (0, 0))))
        offset += n

    scores_p = jnp.stack(sc_parts, axis=1).reshape(B * nl, 1, TILE)
    deltas_p = jnp.stack(dl_parts, axis=1).reshape(B * nl, TILE, 4)
    anchors_p = jnp.stack(an_parts, axis=1).reshape(B * nl, TILE, 4)
    deltas_p = jnp.transpose(deltas_p, (0, 2, 1))
    anchors_p = jnp.transpose(anchors_p, (0, 2, 1))
    packed = jnp.concatenate(
        [scores_p, deltas_p, anchors_p,
         jnp.zeros((B * nl, 7, TILE), jnp.float32)], axis=1)

    out = pl.pallas_call(
        _nms_tile_kernel,
        grid=(B * nl,),
        in_specs=[pl.BlockSpec((1, 16, TILE), lambda i: (i, 0, 0))],
        out_specs=pl.BlockSpec((1, 8, TILE), lambda i: (i, 0, 0)),
        out_shape=jax.ShapeDtypeStruct((B * nl, 8, TILE), jnp.float32),
        scratch_shapes=[
            pltpu.VMEM((TILE // 8, 8, TILE), jnp.float32),
        ],
        compiler_params=pltpu.CompilerParams(
            dimension_semantics=("parallel",)),
    )(packed)

    masked = out[:, 0, :].reshape(B, nl * TILE)
    boxes = jnp.transpose(out[:, 1:5, :].reshape(B, nl, 4, TILE), (0, 1, 3, 2))
    boxes = boxes.reshape(B, nl * TILE, 4)

    top_s, ti2 = jax.lax.top_k(masked, POST_NMS_TOP_N)
    out_b = jnp.take_along_axis(boxes, ti2[:, :, None], axis=1)
    return jnp.concatenate([out_b, top_s[:, :, None]], axis=2)
